# baseline (speedup 1.0000x reference)
import functools

import jax
import jax.numpy as jnp
from jax import lax
from jax.experimental import pallas as pl
from jax.experimental.pallas import tpu as pltpu
from jax.experimental.pallas import tpu_sc as plsc

NUM_BUCKETS = 100000
EMB = 32
BATCH = 16384
MAX_LEN = 100
PLEN = 104

NC = 2
NS = 16
NW = NC * NS
ROWS_PER_W = BATCH // NW
G = 8
NBUF = 2
NCHUNK = ROWS_PER_W // G


def _body(table_hbm, inp_hbm, out_hbm, idx_slab, rows_v, out_v, *sems):
    wid = lax.axis_index("s") * NC + lax.axis_index("c")
    base = wid * ROWS_PER_W

    pltpu.sync_copy(inp_hbm.at[pl.ds(base * PLEN, ROWS_PER_W * PLEN)], idx_slab)

    iota = lax.iota(jnp.int32, 16)
    idx_even = iota * 2
    idx_odd = idx_even + 1
    scale = jnp.float32(1.0 / MAX_LEN)

    def start(c, b):
        pltpu.async_copy(
            table_hbm.at[idx_slab.at[pl.ds(c * (G * PLEN), G * PLEN)]],
            rows_v.at[b],
            sems[b],
        )

    def wait(c, b):
        pltpu.make_async_copy(
            table_hbm.at[idx_slab.at[pl.ds(c * (G * PLEN), G * PLEN)]],
            rows_v.at[b],
            sems[b],
        ).wait()

    def reduce_chunk(c, b):
        for g in range(G):
            acc_e = [jnp.zeros((16,), jnp.float32) for _ in range(4)]
            acc_o = [jnp.zeros((16,), jnp.float32) for _ in range(4)]
            for j in range(PLEN):
                row = rows_v[b, g * PLEN + j, :]
                e, o = plsc.unpack(
                    row,
                    format=plsc.PackFormat.INTERLEAVED,
                    preferred_element_type=jnp.float32,
                )
                acc_e[j % 4] += e
                acc_o[j % 4] += o
            s_e = ((acc_e[0] + acc_e[1]) + (acc_e[2] + acc_e[3])) * scale
            s_o = ((acc_o[0] + acc_o[1]) + (acc_o[2] + acc_o[3])) * scale
            r = c * G + g
            plsc.store_scatter(out_v.at[r], [idx_even], s_e)
            plsc.store_scatter(out_v.at[r], [idx_odd], s_o)

    for b in range(NBUF):
        start(b, b)

    def loop_body(i, _):
        c = i * NBUF
        for b in range(NBUF):
            wait(c + b, b)
            reduce_chunk(c + b, b)
            start(c + b + NBUF, b)
        return 0

    lax.fori_loop(0, NCHUNK // NBUF - 1, loop_body, 0)

    c_last = NCHUNK - NBUF
    for b in range(NBUF):
        wait(c_last + b, b)
        reduce_chunk(c_last + b, b)

    pltpu.sync_copy(out_v, out_hbm.at[pl.ds(base, ROWS_PER_W)])


@functools.partial(jax.jit, donate_argnums=())
def _run(table, inp_flat):
    mesh = plsc.VectorSubcoreMesh(
        core_axis_name="c", subcore_axis_name="s", num_cores=NC, num_subcores=NS
    )
    f = pl.kernel(
        _body,
        out_type=jax.ShapeDtypeStruct((BATCH, EMB), jnp.float32),
        mesh=mesh,
        scratch_types=[
            pltpu.VMEM((ROWS_PER_W * PLEN,), jnp.int32),
            pltpu.VMEM((NBUF, G * PLEN, EMB), jnp.bfloat16),
            pltpu.VMEM((ROWS_PER_W, EMB), jnp.float32),
        ]
        + [pltpu.SemaphoreType.DMA] * NBUF,
        compiler_params=pltpu.CompilerParams(
            use_tc_tiling_on_sc=False, needs_layout_passes=False
        ),
    )
    return f(table, inp_flat)


def kernel(input, embed_weight):
    table_bf = embed_weight.astype(jnp.bfloat16)
    inp_flat = jnp.pad(input, ((0, 0), (0, PLEN - MAX_LEN))).reshape(-1)
    return _run(table_bf, inp_flat)
